# baseline (speedup 1.0000x reference)
import functools

import jax
import jax.numpy as jnp
from jax import lax
from jax.experimental import pallas as pl
from jax.experimental.pallas import tpu as pltpu
from jax.experimental.pallas import tpu_sc as plsc


def _sc_gather(tok_emb, idx):
    B = idx.shape[0]
    D = tok_emb.shape[1]
    info = plsc.get_sparse_core_info()
    NC, NS = info.num_cores, info.num_subcores
    NW = NC * NS
    b_per_w = B // NW
    mesh = plsc.VectorSubcoreMesh(core_axis_name="c", subcore_axis_name="s")

    @functools.partial(
        pl.kernel,
        mesh=mesh,
        out_type=jax.ShapeDtypeStruct((B, D), jnp.float32),
        scratch_types=[
            pltpu.VMEM((b_per_w,), jnp.int32),
            pltpu.VMEM((b_per_w, D), jnp.float32),
            pltpu.SemaphoreType.DMA,
        ],
    )
    def k(table_hbm, idx_hbm, out_hbm, idx_v, rows_v, sem):
        wid = lax.axis_index("s") * NC + lax.axis_index("c")
        base = wid * b_per_w
        pltpu.sync_copy(idx_hbm.at[pl.ds(base, b_per_w)], idx_v)
        pltpu.async_copy(table_hbm.at[idx_v], rows_v, sem).wait()
        pltpu.sync_copy(rows_v, out_hbm.at[pl.ds(base, b_per_w)])

    return k(tok_emb, idx)


def _tc_project(t, pos_emb, W_out, batch, seq, bn):
    M, E = t.shape
    V = W_out.shape[0]
    nv = pl.cdiv(V, bn)

    def body(t_ref, p_ref, w_ref, o_ref):
        w = w_ref[...]
        for b in range(batch):
            h = t_ref[pl.ds(b * seq, seq), :] + p_ref[...]
            o_ref[pl.ds(b * seq, seq), :] = lax.dot_general(
                h, w, (((1,), (1,)), ((), ())),
                preferred_element_type=jnp.float32)

    return pl.pallas_call(
        body,
        grid=(nv,),
        in_specs=[
            pl.BlockSpec((M, E), lambda j: (0, 0)),
            pl.BlockSpec((seq, E), lambda j: (0, 0)),
            pl.BlockSpec((bn, E), lambda j: (j, 0)),
        ],
        out_specs=pl.BlockSpec((M, bn), lambda j: (0, j)),
        out_shape=jax.ShapeDtypeStruct((M, V), jnp.float32),
        compiler_params=pltpu.CompilerParams(
            dimension_semantics=("arbitrary",),
        ),
    )(t, pos_emb, W_out)


def kernel(x, tok_emb, pos_emb, W_out):
    B, S = x.shape
    V, E = W_out.shape
    idx = x.reshape(-1).astype(jnp.int32)
    t = _sc_gather(tok_emb, idx)
    logits = _tc_project(t, pos_emb, W_out, B, S, bn=512)
    return logits.reshape(B, S, V)
